# Initial kernel scaffold; baseline (speedup 1.0000x reference)
import jax
import jax.numpy as jnp
from jax.experimental import pallas as pl


def kernel(x, board_table, pos_table):
    raise NotImplementedError("write your pallas kernel here")


# speedup vs baseline: 3.5799x; 3.5799x over previous
import functools

import jax
import jax.numpy as jnp
from jax.experimental import pallas as pl
from jax.experimental.pallas import tpu as pltpu
from jax.experimental.pallas import tpu_sc as plsc

BOARD_DIM = 42
VOCAB = 3
EMBED_DIM = 64
BATCH = 16384

N_TOK = BATCH * BOARD_DIM
NW = 32
TOK_W = N_TOK // NW
CHUNK = 2688
NCHUNK = TOK_W // CHUNK
K = 128
SUB = CHUNK // K
IDX_ITERS = CHUNK // 16


def _combine_body(board_ref, pos_ref, out_ref):
    out_ref[...] = pos_ref[...][:, None, :] + board_ref[...][None, :, :]


def _build_combined(board_table, pos_table):
    return pl.pallas_call(
        _combine_body,
        out_shape=jax.ShapeDtypeStruct((BOARD_DIM, VOCAB, EMBED_DIM), jnp.float32),
    )(board_table, pos_table)


def _sc_body(pcoef_hbm, x_hbm, comb_hbm, out_hbm,
             x_v, pcoef_v, idx_v, rows_v, gsem):
    wid = jax.lax.axis_index("s") * 2 + jax.lax.axis_index("c")
    base0 = wid * TOK_W
    pltpu.sync_copy(pcoef_hbm, pcoef_v)

    def chunk_body(ch, _):
        base = base0 + ch * CHUNK

        pltpu.sync_copy(x_hbm.at[pl.ds(base, CHUNK)], x_v)

        def idx_body(i, _):
            r = i // (K // 16)
            c = (i % (K // 16)) * 16
            sl = pl.ds(i * 16, 16)
            idx_v[r, pl.ds(c, 16)] = x_v[sl] + pcoef_v[sl]
            return 0

        jax.lax.fori_loop(0, IDX_ITERS, idx_body, 0)

        for s in range(SUB):
            pltpu.async_copy(comb_hbm.at[idx_v.at[s]], rows_v, gsem).wait()
            pltpu.sync_copy(rows_v, out_hbm.at[pl.ds(base + s * K, K)])
        return 0

    jax.lax.fori_loop(0, NCHUNK, chunk_body, 0)


@jax.jit
def kernel(x, board_table, pos_table):
    combined = _build_combined(board_table, pos_table).reshape(VOCAB * BOARD_DIM, EMBED_DIM)
    x_flat = x.reshape(-1).astype(jnp.int32)
    pcoef = (jnp.arange(CHUNK, dtype=jnp.int32) % BOARD_DIM) * VOCAB

    mesh = plsc.VectorSubcoreMesh(core_axis_name="c", subcore_axis_name="s")
    out = pl.kernel(
        _sc_body,
        out_type=jax.ShapeDtypeStruct((N_TOK, EMBED_DIM), jnp.float32),
        mesh=mesh,
        scratch_types=[
            pltpu.VMEM((CHUNK,), jnp.int32),
            pltpu.VMEM((CHUNK,), jnp.int32),
            pltpu.VMEM((SUB, K), jnp.int32),
            pltpu.VMEM((K, EMBED_DIM), jnp.float32),
            pltpu.SemaphoreType.DMA,
        ],
        compiler_params=pltpu.CompilerParams(use_tc_tiling_on_sc=False),
    )(pcoef, x_flat, combined)
    return out.reshape(BATCH, BOARD_DIM, EMBED_DIM)


# speedup vs baseline: 3.6110x; 1.0087x over previous
import functools

import jax
import jax.numpy as jnp
from jax.experimental import pallas as pl
from jax.experimental.pallas import tpu as pltpu
from jax.experimental.pallas import tpu_sc as plsc

BOARD_DIM = 42
VOCAB = 3
EMBED_DIM = 64
BATCH = 16384

N_TOK = BATCH * BOARD_DIM
NW = 32
TOK_W = N_TOK // NW
CHUNK = 2688
NCHUNK = TOK_W // CHUNK
K = 128
SUB = CHUNK // K
IDX_ITERS = CHUNK // 16


def _combine_body(board_ref, pos_ref, out_ref):
    out_ref[...] = pos_ref[...][:, None, :] + board_ref[...][None, :, :]


def _build_combined(board_table, pos_table):
    return pl.pallas_call(
        _combine_body,
        out_shape=jax.ShapeDtypeStruct((BOARD_DIM, VOCAB, EMBED_DIM), jnp.float32),
    )(board_table, pos_table)


NB = 7
GLAG = 3


def _sc_body(pcoef_hbm, x_hbm, comb_hbm, out_hbm,
             x_v, pcoef_v, idx_v, rows_v, gsem, osem):
    wid = jax.lax.axis_index("s") * 2 + jax.lax.axis_index("c")
    base0 = wid * TOK_W
    pltpu.sync_copy(pcoef_hbm, pcoef_v)

    def chunk_body(ch, _):
        base = base0 + ch * CHUNK

        pltpu.sync_copy(x_hbm.at[pl.ds(base, CHUNK)], x_v)

        def idx_body(i, _):
            r = i // (K // 16)
            c = (i % (K // 16)) * 16
            sl = pl.ds(i * 16, 16)
            idx_v[r, pl.ds(c, 16)] = x_v[sl] + pcoef_v[sl]
            return 0

        jax.lax.fori_loop(0, IDX_ITERS, idx_body, 0)

        for t in range(SUB + GLAG):
            if t >= GLAG:
                s = t - GLAG
                b = s % NB
                pltpu.make_async_copy(
                    comb_hbm.at[idx_v.at[s]], rows_v.at[b], gsem.at[b]).wait()
                pltpu.async_copy(
                    rows_v.at[b], out_hbm.at[pl.ds(base + s * K, K)], osem.at[b])
            if t < SUB:
                b = t % NB
                wait_write = pltpu.make_async_copy(
                    rows_v.at[b], out_hbm.at[pl.ds(base + t * K, K)], osem.at[b]).wait
                if t >= NB:
                    wait_write()
                else:
                    pl.when(ch > 0)(wait_write)
                pltpu.async_copy(comb_hbm.at[idx_v.at[t]], rows_v.at[b], gsem.at[b])
        return 0

    jax.lax.fori_loop(0, NCHUNK, chunk_body, 0)

    for b in range(NB):
        pltpu.make_async_copy(
            rows_v.at[b], out_hbm.at[pl.ds(base0, K)], osem.at[b]).wait()


@jax.jit
def kernel(x, board_table, pos_table):
    combined = _build_combined(board_table, pos_table).reshape(VOCAB * BOARD_DIM, EMBED_DIM)
    x_flat = x.reshape(-1).astype(jnp.int32)
    pcoef = (jnp.arange(CHUNK, dtype=jnp.int32) % BOARD_DIM) * VOCAB

    mesh = plsc.VectorSubcoreMesh(core_axis_name="c", subcore_axis_name="s")
    out = pl.kernel(
        _sc_body,
        out_type=jax.ShapeDtypeStruct((N_TOK, EMBED_DIM), jnp.float32),
        mesh=mesh,
        scratch_types=[
            pltpu.VMEM((CHUNK,), jnp.int32),
            pltpu.VMEM((CHUNK,), jnp.int32),
            pltpu.VMEM((SUB, K), jnp.int32),
            pltpu.VMEM((NB, K, EMBED_DIM), jnp.float32),
            pltpu.SemaphoreType.DMA((NB,)),
            pltpu.SemaphoreType.DMA((NB,)),
        ],
        compiler_params=pltpu.CompilerParams(use_tc_tiling_on_sc=False),
    )(pcoef, x_flat, combined)
    return out.reshape(BATCH, BOARD_DIM, EMBED_DIM)


# speedup vs baseline: 3.7216x; 1.0306x over previous
import functools

import jax
import jax.numpy as jnp
from jax.experimental import pallas as pl
from jax.experimental.pallas import tpu as pltpu
from jax.experimental.pallas import tpu_sc as plsc

BOARD_DIM = 42
VOCAB = 3
EMBED_DIM = 64
BATCH = 16384

NW = 32
BW_B = BATCH // NW
NG = BW_B // 16
XBLK = BW_B * BOARD_DIM
TC_PER_W = BW_B // 128
N_TR = BOARD_DIM * 8


def _combine_body(board_ref, pos_ref, out_ref):
    out_ref[...] = pos_ref[...][:, None, :] + board_ref[...][None, :, :]


def _build_combined(board_table, pos_table):
    return pl.pallas_call(
        _combine_body,
        out_shape=jax.ShapeDtypeStruct((BOARD_DIM, VOCAB, EMBED_DIM), jnp.float32),
    )(board_table, pos_table)


def _sc_body(x_hbm, comb_hbm, out_hbm, x_v, comb_v, stage_v, osem):
    wid = jax.lax.axis_index("s") * 2 + jax.lax.axis_index("c")
    pltpu.sync_copy(comb_hbm, comb_v)
    pltpu.sync_copy(x_hbm.at[pl.ds(wid * XBLK, XBLK)], x_v)
    lanes = jax.lax.iota(jnp.int32, 16)
    xg_base = lanes * BOARD_DIM
    tc0 = wid * TC_PER_W

    def halves(t, _):
        for half in range(2):
            p = 2 * t + half
            dst = out_hbm.at[pl.ds(p * 8, 8), pl.ds(tc0, TC_PER_W), :, :]
            wait_prev = pltpu.make_async_copy(stage_v.at[half], dst, osem.at[half]).wait
            pl.when(t > 0)(wait_prev)

            for dc in range(TC_PER_W):
                def fill(gi, _, dc=dc):
                    g = dc * 8 + gi
                    xv = plsc.load_gather(x_v, [xg_base + (g * (16 * BOARD_DIM) + p)])
                    av = xv * EMBED_DIM + (p * (VOCAB * EMBED_DIM))
                    for d in range(EMBED_DIM):
                        val = plsc.load_gather(comb_v, [av + d])
                        stage_v[half, d // 8, dc, d % 8, pl.ds(gi * 16, 16)] = val
                    return 0

                jax.lax.fori_loop(0, NG // TC_PER_W, fill, 0)
            pltpu.async_copy(stage_v.at[half], dst, osem.at[half])
        return 0

    jax.lax.fori_loop(0, BOARD_DIM // 2, halves, 0)

    for half in range(2):
        p = BOARD_DIM - 2 + half
        dst = out_hbm.at[pl.ds(p * 8, 8), pl.ds(tc0, TC_PER_W), :, :]
        pltpu.make_async_copy(stage_v.at[half], dst, osem.at[half]).wait()


@jax.jit
def kernel(x, board_table, pos_table):
    combined = _build_combined(board_table, pos_table).reshape(-1)
    x_flat = x.reshape(-1).astype(jnp.int32)

    mesh = plsc.VectorSubcoreMesh(core_axis_name="c", subcore_axis_name="s")
    out4 = pl.kernel(
        _sc_body,
        out_type=jax.ShapeDtypeStruct((N_TR, 128, 8, 128), jnp.float32),
        mesh=mesh,
        scratch_types=[
            pltpu.VMEM((XBLK,), jnp.int32),
            pltpu.VMEM((VOCAB * BOARD_DIM * EMBED_DIM,), jnp.float32),
            pltpu.VMEM((2, 8, TC_PER_W, 8, 128), jnp.float32),
            pltpu.SemaphoreType.DMA((2,)),
        ],
        compiler_params=pltpu.CompilerParams(
            use_tc_tiling_on_sc=False, needs_layout_passes=False),
    )(x_flat, combined)

    m = out4.transpose(0, 2, 1, 3).reshape(BOARD_DIM * EMBED_DIM, BATCH)
    m = m.reshape(BOARD_DIM, EMBED_DIM, BATCH)
    return jnp.transpose(m, (2, 0, 1))


# speedup vs baseline: 29.6908x; 7.9781x over previous
import functools

import jax
import jax.numpy as jnp
from jax.experimental import pallas as pl
from jax.experimental.pallas import tpu as pltpu
from jax.experimental.pallas import tpu_sc as plsc

BOARD_DIM = 42
VOCAB = 3
EMBED_DIM = 64
BATCH = 16384

NW = 32
BW_B = BATCH // NW
NG = BW_B // 16
XBLK = BW_B * BOARD_DIM
TC_PER_W = BW_B // 128
N_TR = BOARD_DIM * 8


def _combine_body(board_ref, pos_ref, out_ref):
    out_ref[...] = pos_ref[...][:, None, :] + board_ref[...][None, :, :]


def _build_combined(board_table, pos_table):
    return pl.pallas_call(
        _combine_body,
        out_shape=jax.ShapeDtypeStruct((BOARD_DIM, VOCAB, EMBED_DIM), jnp.float32),
    )(board_table, pos_table)


def _sc_body(x_hbm, comb_hbm, out_hbm, x_v, comb_v, xt_v, stage_v, osem):
    wid = jax.lax.axis_index("s") * 2 + jax.lax.axis_index("c")
    pltpu.sync_copy(comb_hbm, comb_v)
    pltpu.sync_copy(x_hbm.at[pl.ds(wid * XBLK, XBLK)], x_v)
    lanes = jax.lax.iota(jnp.int32, 16)
    xg_base = lanes * BOARD_DIM
    tc0 = wid * TC_PER_W

    def halves(t, _):
        for half in range(2):
            p = 2 * t + half
            dst = out_hbm.at[pl.ds(p * 8, 8), pl.ds(tc0, TC_PER_W), :, :]
            wait_prev = pltpu.make_async_copy(stage_v.at[half], dst, osem.at[half]).wait
            pl.when(t > 0)(wait_prev)

            def transpose_x(g, _):
                xv = plsc.load_gather(x_v, [xg_base + (g * (16 * BOARD_DIM) + p)])
                xt_v[pl.ds(g * 16, 16)] = xv
                return 0

            jax.lax.fori_loop(0, NG, transpose_x, 0)
            rows = [comb_v[pl.ds(p * (VOCAB * EMBED_DIM) + v * EMBED_DIM + k * 16, 16)]
                    for v in range(VOCAB) for k in range(4)]

            def splat(v, d):
                row = rows[v * 4 + d // 16]
                idx = jnp.full((16, 1), d % 16, dtype=jnp.int32)
                return jax.lax.gather(
                    row, idx,
                    jax.lax.GatherDimensionNumbers(
                        offset_dims=(), collapsed_slice_dims=(0,),
                        start_index_map=(0,)),
                    (1,),
                    mode=jax.lax.GatherScatterMode.PROMISE_IN_BOUNDS)

            for r in range(8):
                cs = [[splat(v, dr * 8 + r) for v in range(VOCAB)]
                      for dr in range(8)]
                for dc in range(TC_PER_W):
                    def fill(gi, _, r=r, dc=dc, cs=cs):
                        xv = xt_v[pl.ds(dc * 128 + gi * 16, 16)]
                        m1 = xv == 1
                        m2 = xv == 2
                        for dr in range(8):
                            val = jax.lax.select_n(m1, cs[dr][0], cs[dr][1])
                            val = jax.lax.select_n(m2, val, cs[dr][2])
                            stage_v[half, dr, dc, r, pl.ds(gi * 16, 16)] = val
                        return 0

                    jax.lax.fori_loop(0, NG // TC_PER_W, fill, 0)
            pltpu.async_copy(stage_v.at[half], dst, osem.at[half])
        return 0

    jax.lax.fori_loop(0, BOARD_DIM // 2, halves, 0)

    for half in range(2):
        p = BOARD_DIM - 2 + half
        dst = out_hbm.at[pl.ds(p * 8, 8), pl.ds(tc0, TC_PER_W), :, :]
        pltpu.make_async_copy(stage_v.at[half], dst, osem.at[half]).wait()


@jax.jit
def kernel(x, board_table, pos_table):
    combined = _build_combined(board_table, pos_table).reshape(-1)
    x_flat = x.reshape(-1).astype(jnp.int32)

    mesh = plsc.VectorSubcoreMesh(core_axis_name="c", subcore_axis_name="s")
    out4 = pl.kernel(
        _sc_body,
        out_type=jax.ShapeDtypeStruct((N_TR, 128, 8, 128), jnp.float32),
        mesh=mesh,
        scratch_types=[
            pltpu.VMEM((XBLK,), jnp.int32),
            pltpu.VMEM((VOCAB * BOARD_DIM * EMBED_DIM,), jnp.float32),
            pltpu.VMEM((BW_B,), jnp.int32),
            pltpu.VMEM((2, 8, TC_PER_W, 8, 128), jnp.float32),
            pltpu.SemaphoreType.DMA((2,)),
        ],
        compiler_params=pltpu.CompilerParams(
            use_tc_tiling_on_sc=False, needs_layout_passes=False),
    )(x_flat, combined)

    m = out4.transpose(0, 2, 1, 3).reshape(BOARD_DIM * EMBED_DIM, BATCH)
    m = m.reshape(BOARD_DIM, EMBED_DIM, BATCH)
    return jnp.transpose(m, (2, 0, 1))
